# Initial kernel scaffold; baseline (speedup 1.0000x reference)
import jax
import jax.numpy as jnp
from jax.experimental import pallas as pl


def kernel(indices, table):
    raise NotImplementedError("write your pallas kernel here")


# speedup vs baseline: 4.0825x; 4.0825x over previous
import functools

import jax
import jax.numpy as jnp
from jax import lax
from jax.experimental import pallas as pl
from jax.experimental.pallas import tpu as pltpu
from jax.experimental.pallas import tpu_sc as plsc

D = 64
NC, NS = 2, 16
NW = NC * NS
CHUNK = 128


@functools.lru_cache(maxsize=None)
def _make_gather(b_total: int):
    per_w = b_total // NW
    n_chunks = per_w // CHUNK

    mesh = plsc.VectorSubcoreMesh(core_axis_name="c", subcore_axis_name="s")

    @functools.partial(
        pl.kernel,
        out_type=jax.ShapeDtypeStruct((b_total, D), jnp.float32),
        mesh=mesh,
        scratch_types=[
            pltpu.VMEM((n_chunks, CHUNK), jnp.int32),
            pltpu.VMEM((CHUNK, D), jnp.float32),
            pltpu.SemaphoreType.DMA,
        ],
        compiler_params=pltpu.CompilerParams(use_tc_tiling_on_sc=False),
    )
    def gather_kernel(table_hbm, idx_hbm, out_hbm, idx_v, rows_v, sem):
        wid = lax.axis_index("s") * NC + lax.axis_index("c")
        pltpu.sync_copy(idx_hbm.at[wid], idx_v)
        base = wid * per_w

        @pl.loop(0, n_chunks)
        def _(j):
            pltpu.async_copy(table_hbm.at[idx_v.at[j]], rows_v, sem).wait()
            pltpu.sync_copy(rows_v, out_hbm.at[pl.ds(base + j * CHUNK, CHUNK)])

    return gather_kernel


def kernel(indices, table):
    b_total = indices.size
    idx = indices.astype(jnp.int32).reshape(NW, b_total // NW // CHUNK, CHUNK)
    out = _make_gather(b_total)(table, idx)
    return out.reshape(indices.shape + (table.shape[1],))


# speedup vs baseline: 4.6682x; 1.1435x over previous
import functools

import jax
import jax.numpy as jnp
from jax import lax
from jax.experimental import pallas as pl
from jax.experimental.pallas import tpu as pltpu
from jax.experimental.pallas import tpu_sc as plsc

D = 64
NC, NS = 2, 16
NW = NC * NS
CHUNK = 128


GROUP = 5
NBUF = 2


@functools.lru_cache(maxsize=None)
def _make_gather(b_total: int):
    per_w = b_total // NW
    n_chunks = per_w // CHUNK
    n_groups = n_chunks // GROUP
    g_rows = GROUP * CHUNK

    mesh = plsc.VectorSubcoreMesh(core_axis_name="c", subcore_axis_name="s")

    @functools.partial(
        pl.kernel,
        out_type=jax.ShapeDtypeStruct((b_total, D), jnp.float32),
        mesh=mesh,
        scratch_types=[
            pltpu.VMEM((n_chunks, CHUNK), jnp.int32),
            pltpu.VMEM((NBUF, g_rows, D), jnp.float32),
            pltpu.SemaphoreType.DMA((NBUF,)),
            pltpu.SemaphoreType.DMA((NBUF,)),
        ],
        compiler_params=pltpu.CompilerParams(use_tc_tiling_on_sc=False),
    )
    def gather_kernel(table_hbm, idx_hbm, out_hbm, idx_v, rows_v, gsem, wsem):
        wid = lax.axis_index("s") * NC + lax.axis_index("c")
        pltpu.sync_copy(idx_hbm.at[wid], idx_v)
        base = wid * per_w

        def fire_group(g, b):
            for k in range(GROUP):
                pltpu.async_copy(
                    table_hbm.at[idx_v.at[g * GROUP + k]],
                    rows_v.at[b, pl.ds(k * CHUNK, CHUNK)],
                    gsem.at[b],
                )

        def drain_group(b):
            for k in range(GROUP):
                pltpu.make_async_copy(
                    table_hbm.at[idx_v.at[0]],
                    rows_v.at[b, pl.ds(k * CHUNK, CHUNK)],
                    gsem.at[b],
                ).wait()

        for b in range(NBUF):
            fire_group(b, b)

        @pl.loop(0, n_groups, step=NBUF)
        def _(g0):
            for b in range(NBUF):
                g = g0 + b
                drain_group(b)
                pltpu.async_copy(
                    rows_v.at[b],
                    out_hbm.at[pl.ds(base + g * g_rows, g_rows)],
                    wsem.at[b],
                )
                ng = g + NBUF

                @pl.when(ng < n_groups)
                def _():
                    pltpu.make_async_copy(
                        rows_v.at[b],
                        out_hbm.at[pl.ds(base, g_rows)],
                        wsem.at[b],
                    ).wait()
                    fire_group(ng, b)

        for b in range(NBUF):
            pltpu.make_async_copy(
                rows_v.at[b],
                out_hbm.at[pl.ds(base, g_rows)],
                wsem.at[b],
            ).wait()

    return gather_kernel


def kernel(indices, table):
    b_total = indices.size
    idx = indices.astype(jnp.int32).reshape(NW, b_total // NW // CHUNK, CHUNK)
    out = _make_gather(b_total)(table, idx)
    return out.reshape(indices.shape + (table.shape[1],))
